# baseline (speedup 1.0000x reference)
import functools

import jax
import jax.numpy as jnp
from jax import lax
from jax.experimental import pallas as pl
from jax.experimental.pallas import tpu as pltpu
from jax.experimental.pallas import tpu_sc as plsc

EMB = 1024
BATCH = 4
SEQ = 4096
NUM_SEG = 3
NUM_ROWS = BATCH * SEQ
NC = 2
NS = 16
NW = NC * NS
RPW = NUM_ROWS // NW
R0 = 528
R1 = NUM_ROWS // NS - R0
GRP = 16
LAG = 16

_mesh = plsc.VectorSubcoreMesh(core_axis_name="c", subcore_axis_name="s")


@functools.partial(
    pl.kernel,
    mesh=_mesh,
    out_type=jax.ShapeDtypeStruct((NUM_ROWS, EMB), jnp.float32),
    scratch_types=[
        pltpu.VMEM((R0,), jnp.int32),
        pltpu.VMEM((NUM_SEG, EMB), jnp.float32),
        pltpu.SemaphoreType.DMA,
        pltpu.SemaphoreType.DMA,
    ],
)
def _seg_gather(seg_hbm, table_hbm, out_hbm, idx_v, table_v, sem, ssem):
    cid = lax.axis_index("c")
    sid = lax.axis_index("s")
    base = sid * (R0 + R1) + cid * R0
    nrows = jnp.where(cid == 0, R0, R1)
    ng = nrows // GRP
    pltpu.async_copy(seg_hbm.at[pl.ds(base, R0)], idx_v, ssem)
    pltpu.async_copy(table_hbm, table_v, ssem)
    pltpu.make_async_copy(seg_hbm.at[pl.ds(base, R0)], idx_v, ssem).wait()
    pltpu.make_async_copy(table_hbm, table_v, ssem).wait()

    def issue_group(g):
        v = idx_v[pl.ds(g * GRP, GRP)]
        for j in range(GRP):
            pltpu.async_copy(table_v.at[v[j]], out_hbm.at[base + g * GRP + j],
                             sem)

    def wait_group(_g, _):
        pltpu.make_async_copy(out_hbm.at[pl.ds(base, GRP)],
                              out_hbm.at[pl.ds(base, GRP)], sem).wait()
        return 0

    def step(g, _):
        issue_group(g)
        return lax.cond(g >= LAG, lambda: wait_group(g, 0), lambda: 0)

    lax.fori_loop(0, ng, step, 0, unroll=False)

    def tail_wait(g, _):
        return lax.cond(g < jnp.minimum(ng, LAG),
                        lambda: wait_group(g, 0), lambda: 0)

    lax.fori_loop(0, LAG, tail_wait, 0, unroll=False)


def kernel(unused, seg, table):
    del unused
    out = _seg_gather(seg.reshape(NUM_ROWS), table)
    return out.reshape(BATCH, SEQ, EMB)


# speedup vs baseline: 1.0090x; 1.0090x over previous
import functools

import jax
import jax.numpy as jnp
from jax import lax
from jax.experimental import pallas as pl
from jax.experimental.pallas import tpu as pltpu
from jax.experimental.pallas import tpu_sc as plsc

EMB = 1024
BATCH = 4
SEQ = 4096
NUM_SEG = 3
NUM_ROWS = BATCH * SEQ
NC = 2
NS = 16
NW = NC * NS
RPW = NUM_ROWS // NW
GRP = 16
NG = RPW // GRP
LAG = 16

_mesh = plsc.VectorSubcoreMesh(core_axis_name="c", subcore_axis_name="s")


@functools.partial(
    pl.kernel,
    mesh=_mesh,
    out_type=jax.ShapeDtypeStruct((NUM_ROWS, EMB), jnp.float32),
    scratch_types=[
        pltpu.VMEM((RPW,), jnp.int32),
        pltpu.VMEM((NUM_SEG, EMB), jnp.float32),
        pltpu.SemaphoreType.DMA,
        pltpu.SemaphoreType.DMA,
    ],
)
def _seg_gather(seg_hbm, table_hbm, out_hbm, idx_v, table_v, sem, ssem):
    wid = lax.axis_index("s") * NC + lax.axis_index("c")
    base = wid * RPW
    b = wid // (SEQ // RPW)
    col = (wid % (SEQ // RPW)) * RPW

    pltpu.async_copy(seg_hbm.at[b].at[pl.ds(col, RPW)], idx_v, ssem)
    pltpu.async_copy(table_hbm, table_v, ssem)
    pltpu.make_async_copy(seg_hbm.at[b].at[pl.ds(col, RPW)], idx_v, ssem).wait()
    pltpu.make_async_copy(table_hbm, table_v, ssem).wait()

    def issue_group(g):
        v = idx_v[pl.ds(g * GRP, GRP)]
        for j in range(GRP):
            pltpu.async_copy(table_v.at[v[j]], out_hbm.at[base + g * GRP + j],
                             sem)

    def wait_group(_g, _):
        pltpu.make_async_copy(out_hbm.at[pl.ds(base, GRP)],
                              out_hbm.at[pl.ds(base, GRP)], sem).wait()
        return 0

    def step(g, _):
        issue_group(g)
        return lax.cond(g >= LAG, lambda: wait_group(g, 0), lambda: 0)

    lax.fori_loop(0, NG, step, 0, unroll=False)
    lax.fori_loop(0, LAG, wait_group, 0, unroll=False)


def kernel(unused, seg, table):
    del unused
    out = _seg_gather(seg, table)
    return out.reshape(BATCH, SEQ, EMB)
